# baseline (speedup 1.0000x reference)
import jax
import jax.numpy as jnp
from jax.experimental import pallas as pl
from jax.experimental.pallas import tpu as pltpu


def kernel(x, language_id, language_embeddings):
    batch, seq, d = x.shape
    v = language_embeddings.shape[0]
    tile = 1024
    per_batch = seq // tile
    x2 = x.reshape(batch * seq, d)
    tab3 = language_embeddings[:, None, :]
    lid = language_id.astype(jnp.int32)

    def body(x_ref, lid_ref, tab_ref, o_ref):
        i = pl.program_id(0)
        row = lid_ref[i // per_batch]
        o_ref[...] = x_ref[...] + tab_ref[row]

    out2 = pl.pallas_call(
        body,
        grid=(batch * per_batch,),
        in_specs=[
            pl.BlockSpec((tile, d), lambda i: (i, 0)),
            pl.BlockSpec(memory_space=pltpu.SMEM),
            pl.BlockSpec((v, 1, d), lambda i: (0, 0, 0)),
        ],
        out_specs=pl.BlockSpec((tile, d), lambda i: (i, 0)),
        out_shape=jax.ShapeDtypeStruct(x2.shape, x.dtype),
        compiler_params=pltpu.CompilerParams(
            dimension_semantics=("arbitrary",),
        ),
    )(x2, lid, tab3)
    return out2.reshape(batch, seq, d)


# speedup vs baseline: 1.0088x; 1.0088x over previous
import jax
import jax.numpy as jnp
from jax.experimental import pallas as pl
from jax.experimental.pallas import tpu as pltpu


def kernel(x, language_id, language_embeddings):
    batch, seq, d = x.shape
    rows = batch * seq
    x2 = x.reshape(rows, d)
    tab3 = language_embeddings[:, None, :]
    lid = language_id.astype(jnp.int32)

    ch = 512
    nch = rows // ch
    nbuf = 4

    def body(x_hbm, lid_ref, tab_ref, o_hbm, xin, xout, si, so):
        def start_in(c):
            j = c % nbuf
            return pltpu.async_copy(x_hbm.at[pl.ds(c * ch, ch)], xin.at[j], si.at[j])

        def start_out(c):
            j = c % nbuf
            return pltpu.async_copy(xout.at[j], o_hbm.at[pl.ds(c * ch, ch)], so.at[j])

        hin = [None] * nch
        hout = [None] * nch
        for c in range(nbuf):
            hin[c] = start_in(c)
        for c in range(nch):
            j = c % nbuf
            hin[c].wait()
            if c >= nbuf:
                hout[c - nbuf].wait()
            b = (c * ch) // seq
            row = lid_ref[b]
            xout[j] = xin[j] + tab_ref[row]
            hout[c] = start_out(c)
            if c + nbuf < nch:
                hin[c + nbuf] = start_in(c + nbuf)
        for c in range(nch - nbuf, nch):
            hout[c].wait()

    out2 = pl.pallas_call(
        body,
        grid=(),
        in_specs=[
            pl.BlockSpec(memory_space=pltpu.HBM),
            pl.BlockSpec(memory_space=pltpu.SMEM),
            pl.BlockSpec(memory_space=pltpu.VMEM),
        ],
        out_specs=pl.BlockSpec(memory_space=pltpu.HBM),
        out_shape=jax.ShapeDtypeStruct(x2.shape, x.dtype),
        scratch_shapes=[
            pltpu.VMEM((nbuf, ch, d), jnp.float32),
            pltpu.VMEM((nbuf, ch, d), jnp.float32),
            pltpu.SemaphoreType.DMA((nbuf,)),
            pltpu.SemaphoreType.DMA((nbuf,)),
        ],
    )(x2, lid, tab3)
    return out2.reshape(batch, seq, d)


# speedup vs baseline: 1.0375x; 1.0285x over previous
import jax
import jax.numpy as jnp
from jax.experimental import pallas as pl
from jax.experimental.pallas import tpu as pltpu


def kernel(x, language_id, language_embeddings):
    batch, seq, d = x.shape
    v = language_embeddings.shape[0]
    tile = 2048
    per_batch = seq // tile
    x2 = x.reshape(batch * seq, d)
    tab3 = language_embeddings[:, None, :]
    lid = language_id.astype(jnp.int32)

    def body(x_ref, lid_ref, tab_ref, o_ref):
        i = pl.program_id(0)
        row = lid_ref[i // per_batch]
        o_ref[...] = x_ref[...] + tab_ref[row]

    out2 = pl.pallas_call(
        body,
        grid=(batch * per_batch,),
        in_specs=[
            pl.BlockSpec((tile, d), lambda i: (i, 0)),
            pl.BlockSpec(memory_space=pltpu.SMEM),
            pl.BlockSpec((v, 1, d), lambda i: (0, 0, 0)),
        ],
        out_specs=pl.BlockSpec((tile, d), lambda i: (i, 0)),
        out_shape=jax.ShapeDtypeStruct(x2.shape, x.dtype),
        compiler_params=pltpu.CompilerParams(
            dimension_semantics=("arbitrary",),
        ),
    )(x2, lid, tab3)
    return out2.reshape(batch, seq, d)


# speedup vs baseline: 1.0921x; 1.0525x over previous
import jax
import jax.numpy as jnp
from jax.experimental import pallas as pl
from jax.experimental.pallas import tpu as pltpu


def kernel(x, language_id, language_embeddings):
    batch, seq, d = x.shape
    tile = 2048
    if language_id.dtype != jnp.int32:
        language_id = language_id.astype(jnp.int32)

    def body(x_ref, lid_ref, tab_ref, o_ref):
        i = pl.program_id(0)
        row = lid_ref[i]
        o_ref[...] = x_ref[...] + tab_ref[pl.ds(row, 1), :]

    return pl.pallas_call(
        body,
        grid=(batch, seq // tile),
        in_specs=[
            pl.BlockSpec((1, tile, d), lambda i, j: (i, j, 0)),
            pl.BlockSpec(memory_space=pltpu.SMEM),
            pl.BlockSpec(memory_space=pltpu.VMEM),
        ],
        out_specs=pl.BlockSpec((1, tile, d), lambda i, j: (i, j, 0)),
        out_shape=jax.ShapeDtypeStruct(x.shape, x.dtype),
        compiler_params=pltpu.CompilerParams(
            dimension_semantics=("arbitrary", "arbitrary"),
        ),
    )(x, language_id, language_embeddings)


# speedup vs baseline: 1.0925x; 1.0004x over previous
import jax
import jax.numpy as jnp
from jax.experimental import pallas as pl
from jax.experimental.pallas import tpu as pltpu


def kernel(x, language_id, language_embeddings):
    batch, seq, d = x.shape
    tile = 2048
    if language_id.dtype != jnp.int32:
        language_id = language_id.astype(jnp.int32)

    def body(x_ref, lid_ref, tab_ref, o_ref):
        i = pl.program_id(0)
        row = lid_ref[i]
        o_ref[...] = x_ref[...] + tab_ref[pl.ds(row, 1), :]

    return pl.pallas_call(
        body,
        grid=(batch, seq // tile),
        in_specs=[
            pl.BlockSpec((1, tile, d), lambda i, j: (i, j, 0)),
            pl.BlockSpec(memory_space=pltpu.SMEM),
            pl.BlockSpec(memory_space=pltpu.VMEM),
        ],
        out_specs=pl.BlockSpec((1, tile, d), lambda i, j: (i, j, 0)),
        out_shape=jax.ShapeDtypeStruct(x.shape, x.dtype),
        compiler_params=pltpu.CompilerParams(
            dimension_semantics=("parallel", "arbitrary"),
        ),
    )(x, language_id, language_embeddings)
